# baseline (speedup 1.0000x reference)
import functools

import jax
import jax.numpy as jnp
from jax import lax
from jax.experimental import pallas as pl
from jax.experimental.pallas import tpu as pltpu
from jax.experimental.pallas import tpu_sc as plsc

BATCH = 4096
SEQ = 200
D = 64
N = BATCH * SEQ
NC = 2
NS = 16
NW = NC * NS
PER_W = N // NW
C = 128
CHUNKS = PER_W // C
NBUF = 8
PF = 4
PROBE_ADD = True
PROBE_SCATTER = True


def _pe_table():
    pos = jnp.arange(SEQ, dtype=jnp.float32)[:, None]
    i = jnp.arange(0, D, 2, dtype=jnp.float32)
    div = jnp.exp(-jnp.log(10000.0) * i / D)
    pe = jnp.zeros((SEQ, D), dtype=jnp.float32)
    pe = pe.at[:, 0::2].set(jnp.sin(pos * div))
    pe = pe.at[:, 1::2].set(jnp.cos(pos * div))
    return jnp.concatenate([pe, pe], axis=0)


def _sc_embed(table, idx, pe2):
    mesh = plsc.VectorSubcoreMesh(
        core_axis_name="c", subcore_axis_name="s",
        num_cores=NC, num_subcores=NS)

    @functools.partial(
        pl.kernel,
        out_type=jax.ShapeDtypeStruct((N, D), jnp.float32),
        mesh=mesh,
        compiler_params=pltpu.CompilerParams(use_tc_tiling_on_sc=False),
        scratch_types=[
            pltpu.VMEM((2 * SEQ, D), jnp.float32),
            pltpu.VMEM((CHUNKS, C), jnp.int32),
        ] + [pltpu.VMEM((C, D), jnp.float32) for _ in range(NBUF)]
          + [pltpu.SemaphoreType.DMA for _ in range(2 * NBUF)],
    )
    def body(table_hbm, idx_hbm, pe_hbm, out_hbm,
             pe_v, idx_v,
             r0, r1, r2, r3, r4, r5, r6, r7,
             g0s, g1s, g2s, g3s, g4s, g5s, g6s, g7s,
             s0s, s1s, s2s, s3s, s4s, s5s, s6s, s7s):
        rows = (r0, r1, r2, r3, r4, r5, r6, r7)
        gsem = (g0s, g1s, g2s, g3s, g4s, g5s, g6s, g7s)
        ssem = (s0s, s1s, s2s, s3s, s4s, s5s, s6s, s7s)
        wid = lax.axis_index("s") * NC + lax.axis_index("c")
        base = wid * PER_W

        pltpu.sync_copy(pe_hbm, pe_v)
        pltpu.sync_copy(idx_hbm.at[wid], idx_v)

        def issue_gather(h, v):
            for k in range(C // 16):
                iv = idx_v[h, pl.ds(k * 16, 16)]
                pltpu.async_copy(
                    table_hbm.at[iv], rows[v].at[pl.ds(k * 16, 16)], gsem[v])

        for b in range(PF):
            issue_gather(b, b)

        def outer(i, carry):
            for b in range(NBUF):
                g = i * NBUF + b
                pltpu.make_async_copy(
                    out_hbm.at[pl.ds(base, C)], rows[b], gsem[b]).wait()
                p0 = lax.rem(g * C, SEQ)

                def add_body(jj, _, b=b, p0=p0):
                    for r in range(8):
                        j = jj * 8 + r
                        p = p0 + j
                        for dd in range(4):
                            sl = pl.ds(dd * 16, 16)
                            rows[b][j, sl] = rows[b][j, sl] + pe_v[p, sl]
                    return 0

                if PROBE_ADD:
                    lax.fori_loop(0, C // 8, add_body, 0)
                if PROBE_SCATTER:
                    pltpu.async_copy(
                        rows[b], out_hbm.at[pl.ds(base + g * C, C)], ssem[b])
                h = g + PF
                v = (b + PF) % NBUF

                if PROBE_SCATTER:
                    @pl.when((h >= NBUF) & (h < CHUNKS))
                    def _(v=v):
                        pltpu.make_async_copy(
                            rows[v], out_hbm.at[pl.ds(base, C)], ssem[v]).wait()

                @pl.when(h < CHUNKS)
                def _(v=v, h=h):
                    issue_gather(h, v)
            return carry

        lax.fori_loop(0, CHUNKS // NBUF, outer, 0)

        if PROBE_SCATTER:
            for b in range(NBUF):
                pltpu.make_async_copy(
                    rows[b], out_hbm.at[pl.ds(base, C)], ssem[b]).wait()

    return body(table, idx, pe2)


def kernel(x, table):
    idx = x.astype(jnp.int32).reshape(NW, CHUNKS, C)
    out = _sc_embed(table, idx, _pe_table())
    return out.reshape(BATCH, SEQ, D)


# speedup vs baseline: 1.3133x; 1.3133x over previous
import functools

import jax
import jax.numpy as jnp
from jax import lax
from jax.experimental import pallas as pl
from jax.experimental.pallas import tpu as pltpu
from jax.experimental.pallas import tpu_sc as plsc

BATCH = 4096
SEQ = 200
D = 64
N = BATCH * SEQ
NC = 2
NS = 16
NW = NC * NS
PER_W = N // NW
BPW = BATCH // NW
C = 2 * SEQ
CHUNKS = PER_W // C
NBUF = 3
NG = C // 16


def _pe_table():
    pos = jnp.arange(SEQ, dtype=jnp.float32)[:, None]
    i = jnp.arange(0, D, 2, dtype=jnp.float32)
    div = jnp.exp(-jnp.log(10000.0) * i / D)
    pe = jnp.zeros((SEQ, D), dtype=jnp.float32)
    pe = pe.at[:, 0::2].set(jnp.sin(pos * div))
    pe = pe.at[:, 1::2].set(jnp.cos(pos * div))
    return jnp.concatenate([pe, pe], axis=0)


def _sc_embed(table, idx, pe2):
    mesh = plsc.VectorSubcoreMesh(
        core_axis_name="c", subcore_axis_name="s",
        num_cores=NC, num_subcores=NS)

    @functools.partial(
        pl.kernel,
        out_type=jax.ShapeDtypeStruct((BATCH, SEQ, D), jnp.float32),
        mesh=mesh,
        compiler_params=pltpu.CompilerParams(use_tc_tiling_on_sc=False),
        scratch_types=[
            pltpu.VMEM((2 * SEQ, D), jnp.float32),
            pltpu.VMEM((CHUNKS, C), jnp.int32),
        ] + [pltpu.VMEM((C, D), jnp.float32) for _ in range(NBUF)]
          + [pltpu.SemaphoreType.DMA for _ in range(2 * NBUF)],
    )
    def body(table_hbm, idx_hbm, pe_hbm, out_hbm,
             pe_v, idx_v, r0, r1, r2,
             g0s, g1s, g2s, s0s, s1s, s2s):
        rows = (r0, r1, r2)
        gsem = (g0s, g1s, g2s)
        ssem = (s0s, s1s, s2s)
        wid = lax.axis_index("s") * NC + lax.axis_index("c")
        bbase = wid * BPW

        pltpu.sync_copy(pe_hbm, pe_v)
        pltpu.sync_copy(idx_hbm.at[wid], idx_v)

        def issue_gather(h, v):
            for k in range(NG):
                iv = idx_v[h, pl.ds(k * 16, 16)]
                pltpu.async_copy(
                    table_hbm.at[iv], rows[v].at[pl.ds(k * 16, 16)], gsem[v])

        issue_gather(0, 0)

        def outer(i, carry):
            for b in range(NBUF):
                g = i * NBUF + b
                pltpu.make_async_copy(
                    table_hbm.at[pl.ds(0, C)], rows[b], gsem[b]).wait()
                h = g + 1
                v = (b + 1) % NBUF

                @pl.when((h >= NBUF) & (h < CHUNKS))
                def _(v=v):
                    for half in range(2):
                        pltpu.make_async_copy(
                            rows[v].at[pl.ds(0, SEQ)], out_hbm.at[bbase],
                            ssem[v]).wait()

                @pl.when(h < CHUNKS)
                def _(v=v, h=h):
                    issue_gather(h, v)

                def add_body(jj, _, b=b):
                    for r in range(8):
                        j = jj * 8 + r
                        for dd in range(4):
                            sl = pl.ds(dd * 16, 16)
                            plsc.addupdate(rows[b].at[j, sl], pe_v[j, sl])
                    return 0

                lax.fori_loop(0, C // 8, add_body, 0)
                for half in range(2):
                    pltpu.async_copy(
                        rows[b].at[pl.ds(half * SEQ, SEQ)],
                        out_hbm.at[bbase + 2 * g + half], ssem[b])
            return carry

        lax.fori_loop(0, (CHUNKS - 1) // NBUF, outer, 0)

        gl = CHUNKS - 1
        bl = gl % NBUF
        pltpu.make_async_copy(
            table_hbm.at[pl.ds(0, C)], rows[bl], gsem[bl]).wait()

        def add_last(jj, _):
            for r in range(8):
                j = jj * 8 + r
                for dd in range(4):
                    sl = pl.ds(dd * 16, 16)
                    plsc.addupdate(rows[bl].at[j, sl], pe_v[j, sl])
            return 0

        lax.fori_loop(0, C // 8, add_last, 0)
        for half in range(2):
            pltpu.async_copy(
                rows[bl].at[pl.ds(half * SEQ, SEQ)],
                out_hbm.at[bbase + 2 * gl + half], ssem[bl])

        for b in range(NBUF):
            for half in range(2):
                pltpu.make_async_copy(
                    rows[b].at[pl.ds(0, SEQ)], out_hbm.at[bbase],
                    ssem[b]).wait()

    return body(table, idx, pe2)


def kernel(x, table):
    idx = x.astype(jnp.int32).reshape(NW, CHUNKS, C)
    return _sc_embed(table, idx, _pe_table())


# speedup vs baseline: 1.5778x; 1.2014x over previous
import functools

import jax
import jax.numpy as jnp
from jax import lax
from jax.experimental import pallas as pl
from jax.experimental.pallas import tpu as pltpu
from jax.experimental.pallas import tpu_sc as plsc

BATCH = 4096
SEQ = 200
D = 64
NC = 2
NS = 16
NW = NC * NS
BPW = BATCH // NW
C = SEQ
NBUF = 4
NG = 13
GBYTES_MAIN = 192


def _pe_table():
    pos = jnp.arange(SEQ, dtype=jnp.float32)[:, None]
    i = jnp.arange(0, D, 2, dtype=jnp.float32)
    div = jnp.exp(-jnp.log(10000.0) * i / D)
    pe = jnp.zeros((SEQ, D), dtype=jnp.float32)
    pe = pe.at[:, 0::2].set(jnp.sin(pos * div))
    pe = pe.at[:, 1::2].set(jnp.cos(pos * div))
    return jnp.pad(pe, ((0, 0), (0, 2 * D - D)))


def _sc_embed(table_p, idx, pe_p):
    mesh = plsc.VectorSubcoreMesh(
        core_axis_name="c", subcore_axis_name="s",
        num_cores=NC, num_subcores=NS)

    @functools.partial(
        pl.kernel,
        out_type=jax.ShapeDtypeStruct((BATCH, SEQ, 2 * D), jnp.float32),
        mesh=mesh,
        compiler_params=pltpu.CompilerParams(use_tc_tiling_on_sc=True),
        scratch_types=[
            pltpu.VMEM((SEQ, 2 * D), jnp.float32),
        ] + [pltpu.VMEM((C,), jnp.int32) for _ in range(NBUF)]
          + [pltpu.VMEM((C, 2 * D), jnp.float32) for _ in range(NBUF)]
          + [pltpu.SemaphoreType.DMA for _ in range(3 * NBUF)],
    )
    def body(table_hbm, idx_hbm, pe_hbm, out_hbm,
             pe_v, i0, i1, i2, i3, r0, r1, r2, r3,
             gi0, gi1, gi2, gi3, g0s, g1s, g2s, g3s, s0s, s1s, s2s, s3s):
        idxs = (i0, i1, i2, i3)
        rows = (r0, r1, r2, r3)
        isem = (gi0, gi1, gi2, gi3)
        gsem = (g0s, g1s, g2s, g3s)
        ssem = (s0s, s1s, s2s, s3s)
        wid = lax.axis_index("s") * NC + lax.axis_index("c")
        bbase = wid * BPW

        pltpu.sync_copy(pe_hbm, pe_v)

        def issue_gather(v):
            for k in range(NG):
                o = min(k * 16, C - 16)
                iv = idxs[v][pl.ds(o, 16)]
                pltpu.async_copy(
                    table_hbm.at[iv], rows[v].at[pl.ds(o, 16)], gsem[v])

        for h in range(NBUF):
            pltpu.async_copy(idx_hbm.at[wid, h], idxs[h], isem[h])
        pltpu.make_async_copy(idx_hbm.at[wid, 0], idxs[0], isem[0]).wait()
        issue_gather(0)

        def outer(i, carry):
            for b in range(NBUF):
                g = i * NBUF + b
                vb = (b + 1) % NBUF
                pltpu.make_async_copy(
                    table_hbm.at[pl.ds(0, GBYTES_MAIN)],
                    rows[b].at[pl.ds(0, GBYTES_MAIN)], gsem[b]).wait()
                pltpu.make_async_copy(
                    table_hbm.at[pl.ds(0, 16)],
                    rows[b].at[pl.ds(0, 16)], gsem[b]).wait()
                h = g + 1

                @pl.when(g + NBUF < BPW)
                def _(b=b, g=g):
                    pltpu.async_copy(
                        idx_hbm.at[wid, g + NBUF], idxs[b], isem[b])

                @pl.when((h >= NBUF) & (h < BPW))
                def _(vb=vb):
                    pltpu.make_async_copy(
                        rows[vb], out_hbm.at[bbase], ssem[vb]).wait()

                @pl.when(h < BPW)
                def _(vb=vb, h=h):
                    pltpu.make_async_copy(
                        idx_hbm.at[wid, 0], idxs[vb], isem[vb]).wait()
                    issue_gather(vb)

                def add_body(jj, _, b=b):
                    for r in range(8):
                        j = jj * 8 + r
                        for dd in range(4):
                            sl = pl.ds(dd * 16, 16)
                            plsc.addupdate(rows[b].at[j, sl], pe_v[j, sl])
                    return 0

                lax.fori_loop(0, C // 8, add_body, 0)
                pltpu.async_copy(rows[b], out_hbm.at[bbase + g], ssem[b])
            return carry

        lax.fori_loop(0, BPW // NBUF, outer, 0)

        for g in range(BPW - NBUF + 1, BPW):
            b = g % NBUF
            pltpu.make_async_copy(
                rows[b], out_hbm.at[bbase], ssem[b]).wait()

    return body(table_p, idx, pe_p)


def kernel(x, table):
    idx = x.astype(jnp.int32).reshape(NW, BPW, SEQ)
    table_p = jnp.pad(table, ((0, 0), (0, 2 * D - D)))
    return _sc_embed(table_p, idx, _pe_table())[..., :D]
